# baseline (speedup 1.0000x reference)
import functools

import jax
import jax.numpy as jnp
from jax import lax
from jax.experimental import pallas as pl
from jax.experimental.pallas import tpu as pltpu
from jax.experimental.pallas import tpu_sc as plsc

_VOCAB = 100000
_DIM = 128
_BATCH = 4096
_HIST = 200

_ROWS_PER_BLK = 25000
_NUM_BLKS = _VOCAB // _ROWS_PER_BLK


def _tc_matvec_body(tab_ref, wrep_ref, eye_ref, b_ref, out_ref):
    s = jnp.dot(tab_ref[...], wrep_ref[...], preferred_element_type=jnp.float32)
    b = b_ref[0, 0]
    n_full = _ROWS_PER_BLK // _DIM
    for g in range(n_full):
        seg = s[g * _DIM:(g + 1) * _DIM, :] * eye_ref[...]
        col = jnp.sum(seg, axis=0) + b
        out_ref[0, 0, pl.ds(g * _DIM, _DIM)] = col
    tail = _ROWS_PER_BLK - n_full * _DIM
    if tail:
        seg = s[n_full * _DIM:, :] * eye_ref[:tail, :]
        col = jnp.sum(seg, axis=0) + b
        out_ref[0, 0, pl.ds(n_full * _DIM, tail)] = col[:tail]


def _tc_matvec(emb_table, fc_w, fc_b):
    w_rep = jnp.broadcast_to(fc_w.reshape(_DIM, 1), (_DIM, _DIM))
    eye = jnp.eye(_DIM, dtype=jnp.float32)
    b2 = fc_b.reshape(1, 1)
    out = pl.pallas_call(
        _tc_matvec_body,
        grid=(_NUM_BLKS,),
        in_specs=[
            pl.BlockSpec((_ROWS_PER_BLK, _DIM), lambda i: (i, 0)),
            pl.BlockSpec((_DIM, _DIM), lambda i: (0, 0)),
            pl.BlockSpec((_DIM, _DIM), lambda i: (0, 0)),
            pl.BlockSpec(memory_space=pltpu.SMEM),
        ],
        out_specs=pl.BlockSpec((1, 1, _ROWS_PER_BLK), lambda i: (i, 0, 0)),
        out_shape=jax.ShapeDtypeStruct((_NUM_BLKS, 1, _ROWS_PER_BLK), jnp.float32),
    )(emb_table, w_rep, eye, b2)
    return out.reshape(_VOCAB)


def _make_sc_pool():
    info = plsc.get_sparse_core_info()
    nc, ns, lanes = info.num_cores, info.num_subcores, info.num_lanes
    nw = nc * ns
    rows_per_w = _BATCH // nw
    chunks = rows_per_w // lanes

    mesh = plsc.VectorSubcoreMesh(core_axis_name="c", subcore_axis_name="s")

    @functools.partial(
        pl.kernel,
        mesh=mesh,
        compiler_params=pltpu.CompilerParams(needs_layout_passes=False),
        out_type=jax.ShapeDtypeStruct((_BATCH,), jnp.float32),
        scratch_types=[
            pltpu.VMEM((_VOCAB,), jnp.float32),
            pltpu.VMEM((lanes * _HIST,), jnp.int32),
            pltpu.VMEM((rows_per_w,), jnp.float32),
        ],
    )
    def sc_pool(t_hbm, x_hbm, out_hbm, t_v, xb_v, out_v):
        wid = lax.axis_index("s") * nc + lax.axis_index("c")
        base = wid * rows_per_w
        pltpu.sync_copy(t_hbm, t_v)
        row_off = lax.iota(jnp.int32, lanes) * _HIST
        for c in range(chunks):
            pltpu.sync_copy(
                x_hbm.at[pl.ds((base + c * lanes) * _HIST, lanes * _HIST)], xb_v)

            def body(j, acc):
                xv = plsc.load_gather(xb_v, [row_off + j])
                tv = plsc.load_gather(t_v, [xv])
                return acc + tv

            acc = lax.fori_loop(0, _HIST, body, jnp.zeros((lanes,), jnp.float32),
                                unroll=8)
            z = acc * (1.0 / _HIST)
            out_v[pl.ds(c * lanes, lanes)] = 1.0 / (1.0 + jnp.exp(-z))
        pltpu.sync_copy(out_v, out_hbm.at[pl.ds(base, rows_per_w)])

    return sc_pool


def kernel(x, emb_table, fc_w, fc_b):
    t = _tc_matvec(emb_table, fc_w, fc_b)
    pool = _make_sc_pool()
    out = pool(t, x.astype(jnp.int32).reshape(-1))
    return out.reshape(_BATCH, 1)


# speedup vs baseline: 1.0187x; 1.0187x over previous
import functools

import jax
import jax.numpy as jnp
from jax import lax
from jax.experimental import pallas as pl
from jax.experimental.pallas import tpu as pltpu
from jax.experimental.pallas import tpu_sc as plsc

_VOCAB = 100000
_DIM = 128
_BATCH = 4096
_HIST = 200

_ROWS_PER_BLK = 10000
_NUM_BLKS = _VOCAB // _ROWS_PER_BLK


def _tc_matvec_body(tab_ref, wrep_ref, eye_ref, b_ref, out_ref):
    s = jnp.dot(tab_ref[...], wrep_ref[...], preferred_element_type=jnp.float32)
    b = b_ref[0, 0]
    n_full = _ROWS_PER_BLK // _DIM
    for g in range(n_full):
        seg = s[g * _DIM:(g + 1) * _DIM, :] * eye_ref[...]
        col = jnp.sum(seg, axis=0) + b
        out_ref[0, 0, pl.ds(g * _DIM, _DIM)] = col
    tail = _ROWS_PER_BLK - n_full * _DIM
    if tail:
        seg = s[n_full * _DIM:, :] * eye_ref[:tail, :]
        col = jnp.sum(seg, axis=0) + b
        out_ref[0, 0, pl.ds(n_full * _DIM, tail)] = col[:tail]


def _tc_matvec(emb_table, fc_w, fc_b):
    w_rep = jnp.broadcast_to(fc_w.reshape(_DIM, 1), (_DIM, _DIM))
    eye = jnp.eye(_DIM, dtype=jnp.float32)
    b2 = fc_b.reshape(1, 1)
    out = pl.pallas_call(
        _tc_matvec_body,
        grid=(_NUM_BLKS,),
        in_specs=[
            pl.BlockSpec((_ROWS_PER_BLK, _DIM), lambda i: (i, 0)),
            pl.BlockSpec((_DIM, _DIM), lambda i: (0, 0)),
            pl.BlockSpec((_DIM, _DIM), lambda i: (0, 0)),
            pl.BlockSpec(memory_space=pltpu.SMEM),
        ],
        out_specs=pl.BlockSpec((1, 1, _ROWS_PER_BLK), lambda i: (i, 0, 0)),
        out_shape=jax.ShapeDtypeStruct((_NUM_BLKS, 1, _ROWS_PER_BLK), jnp.float32),
    )(emb_table, w_rep, eye, b2)
    return out.reshape(_VOCAB)


def _make_sc_pool():
    info = plsc.get_sparse_core_info()
    nc, ns, lanes = info.num_cores, info.num_subcores, info.num_lanes
    nw = nc * ns
    rows_per_w = _BATCH // nw
    chunks = rows_per_w // lanes

    mesh = plsc.VectorSubcoreMesh(core_axis_name="c", subcore_axis_name="s")

    @functools.partial(
        pl.kernel,
        mesh=mesh,
        compiler_params=pltpu.CompilerParams(needs_layout_passes=False),
        out_type=jax.ShapeDtypeStruct((_BATCH,), jnp.float32),
        scratch_types=[
            pltpu.VMEM((_VOCAB,), jnp.float32),
            pltpu.VMEM((lanes * _HIST,), jnp.int32),
            pltpu.VMEM((rows_per_w,), jnp.float32),
        ],
    )
    def sc_pool(t_hbm, x_hbm, out_hbm, t_v, xb_v, out_v):
        wid = lax.axis_index("s") * nc + lax.axis_index("c")
        base = wid * rows_per_w
        pltpu.sync_copy(t_hbm, t_v)
        row_off = lax.iota(jnp.int32, lanes) * _HIST
        for c in range(chunks):
            pltpu.sync_copy(
                x_hbm.at[pl.ds((base + c * lanes) * _HIST, lanes * _HIST)], xb_v)

            def body(j, acc):
                xv = plsc.load_gather(xb_v, [row_off + j])
                tv = plsc.load_gather(t_v, [xv])
                return acc + tv

            acc = lax.fori_loop(0, _HIST, body, jnp.zeros((lanes,), jnp.float32),
                                unroll=8)
            z = acc * (1.0 / _HIST)
            out_v[pl.ds(c * lanes, lanes)] = 1.0 / (1.0 + jnp.exp(-z))
        pltpu.sync_copy(out_v, out_hbm.at[pl.ds(base, rows_per_w)])

    return sc_pool


def kernel(x, emb_table, fc_w, fc_b):
    t = _tc_matvec(emb_table, fc_w, fc_b)
    pool = _make_sc_pool()
    out = pool(t, x.astype(jnp.int32).reshape(-1))
    return out.reshape(_BATCH, 1)


# speedup vs baseline: 1.1117x; 1.0913x over previous
import functools

import jax
import jax.numpy as jnp
from jax import lax
from jax.experimental import pallas as pl
from jax.experimental.pallas import tpu as pltpu
from jax.experimental.pallas import tpu_sc as plsc

_VOCAB = 100000
_DIM = 128
_BATCH = 4096
_HIST = 200

_ROWS_PER_BLK = 10000
_NUM_BLKS = _VOCAB // _ROWS_PER_BLK


def _tc_matvec_body(tab_ref, wrep_ref, eye_ref, b_ref, out_ref):
    s = jnp.dot(tab_ref[...], wrep_ref[...], preferred_element_type=jnp.float32)
    b = b_ref[0, 0]
    n_full = _ROWS_PER_BLK // _DIM
    for g in range(n_full):
        seg = s[g * _DIM:(g + 1) * _DIM, :] * eye_ref[...]
        col = jnp.sum(seg, axis=0) + b
        out_ref[0, 0, pl.ds(g * _DIM, _DIM)] = col
    tail = _ROWS_PER_BLK - n_full * _DIM
    if tail:
        seg = s[n_full * _DIM:, :] * eye_ref[:tail, :]
        col = jnp.sum(seg, axis=0) + b
        out_ref[0, 0, pl.ds(n_full * _DIM, tail)] = col[:tail]


def _tc_matvec(emb_table, fc_w, fc_b):
    w_rep = jnp.broadcast_to(fc_w.reshape(_DIM, 1), (_DIM, _DIM))
    eye = jnp.eye(_DIM, dtype=jnp.float32)
    b2 = fc_b.reshape(1, 1)
    out = pl.pallas_call(
        _tc_matvec_body,
        grid=(_NUM_BLKS,),
        in_specs=[
            pl.BlockSpec((_ROWS_PER_BLK, _DIM), lambda i: (i, 0)),
            pl.BlockSpec((_DIM, _DIM), lambda i: (0, 0)),
            pl.BlockSpec((_DIM, _DIM), lambda i: (0, 0)),
            pl.BlockSpec(memory_space=pltpu.SMEM),
        ],
        out_specs=pl.BlockSpec((1, 1, _ROWS_PER_BLK), lambda i: (i, 0, 0)),
        out_shape=jax.ShapeDtypeStruct((_NUM_BLKS, 1, _ROWS_PER_BLK), jnp.float32),
    )(emb_table, w_rep, eye, b2)
    return out.reshape(_VOCAB)


def _make_sc_pool():
    info = plsc.get_sparse_core_info()
    nc, ns, lanes = info.num_cores, info.num_subcores, info.num_lanes
    nw = nc * ns
    rows_per_w = _BATCH // nw
    chunks = rows_per_w // lanes

    mesh = plsc.VectorSubcoreMesh(core_axis_name="c", subcore_axis_name="s")

    @functools.partial(
        pl.kernel,
        mesh=mesh,
        compiler_params=pltpu.CompilerParams(needs_layout_passes=False),
        out_type=jax.ShapeDtypeStruct((_BATCH,), jnp.float32),
        scratch_types=[
            pltpu.VMEM((_VOCAB,), jnp.float32),
            pltpu.VMEM((rows_per_w * _HIST,), jnp.int32),
            pltpu.VMEM((rows_per_w,), jnp.float32),
            pltpu.SemaphoreType.DMA,
            pltpu.SemaphoreType.DMA,
        ],
    )
    def sc_pool(t_hbm, x_hbm, out_hbm, t_v, xb_v, out_v, sem_t, sem_x):
        wid = lax.axis_index("s") * nc + lax.axis_index("c")
        base = wid * rows_per_w
        cp_x = pltpu.async_copy(
            x_hbm.at[pl.ds(base * _HIST, rows_per_w * _HIST)], xb_v, sem_x)
        cp_t = pltpu.async_copy(t_hbm, t_v, sem_t)
        cp_x.wait()
        cp_t.wait()
        row_off = lax.iota(jnp.int32, lanes) * _HIST

        def chunk_body(c, _):
            coff = c * (lanes * _HIST)

            def body(j, acc):
                xv = plsc.load_gather(xb_v, [coff + row_off + j])
                tv = plsc.load_gather(t_v, [xv])
                return acc + tv

            acc = lax.fori_loop(0, _HIST, body, jnp.zeros((lanes,), jnp.float32),
                                unroll=8)
            z = acc * (1.0 / _HIST)
            out_v[pl.ds(c * lanes, lanes)] = 1.0 / (1.0 + jnp.exp(-z))
            return 0

        lax.fori_loop(0, chunks, chunk_body, 0)
        pltpu.sync_copy(out_v, out_hbm.at[pl.ds(base, rows_per_w)])

    return sc_pool


def kernel(x, emb_table, fc_w, fc_b):
    t = _tc_matvec(emb_table, fc_w, fc_b)
    pool = _make_sc_pool()
    out = pool(t, x.astype(jnp.int32).reshape(-1))
    return out.reshape(_BATCH, 1)
